# baseline (speedup 1.0000x reference)
import jax
import jax.numpy as jnp
import numpy as np
from jax import lax
from jax.experimental import pallas as pl
from jax.experimental.pallas import tpu as pltpu
from jax.experimental.pallas import tpu_sc as plsc

V = 50000
P = 100000
D = 128
NV = 4
NC = 32
EPS = 1e-5

NW = 32
PPAD = 102400
CHUNK = 128
CHTOT = 50
CH0 = 16
TW = 48
BN = NC
TCOL = NC + 8

NB = 10
VB = V // NB


def _group_avg_matrix(C, G, Vn):
    gid = np.arange(C) // (C // G)
    A = (gid[:, None] == gid[None, :]).astype(np.float32) / (Vn * (C // G))
    return jnp.asarray(A)


def _matmul_t(x, W):
    return lax.dot_general(x, W, (((1,), (1,)), ((), ())),
                           preferred_element_type=jnp.float32)


def _finalize_stats(acc, A):
    meanb = jnp.dot(acc[0:1, :], A, preferred_element_type=jnp.float32)
    ex2b = jnp.dot(acc[1:2, :], A, preferred_element_type=jnp.float32)
    rstd = lax.rsqrt(ex2b - meanb * meanb + EPS)
    return jnp.concatenate([meanb, rstd], axis=0)


def _accum_stats(acc_ref, y):
    i = pl.program_id(0)

    @pl.when(i == 0)
    def _():
        acc_ref[...] = jnp.zeros_like(acc_ref)

    acc_ref[...] += jnp.concatenate(
        [jnp.sum(y, axis=0, keepdims=True),
         jnp.sum(y * y, axis=0, keepdims=True)], axis=0)


def _gn_apply(x, st_ref, g_ref, b_ref):
    xn = (x - st_ref[0:1, :]) * st_ref[1:2, :]
    return jnp.maximum(xn * g_ref[...] + b_ref[...], 0.0)


def _tc_stats0(lv_ref, A_ref, st_ref, acc):
    _accum_stats(acc, lv_ref[...])

    @pl.when(pl.program_id(0) == NB - 1)
    def _():
        st_ref[...] = _finalize_stats(acc[...], A_ref[...])


def _tc_stage0(lv_ref, st_ref, g_ref, b_ref, W_ref, cls_W_ref, A_ref,
               x0_ref, lvcls_ref, st1_ref, acc):
    lv = lv_ref[...]
    lvcls_ref[...] = _matmul_t(lv, cls_W_ref[...])
    y = _matmul_t(_gn_apply(lv, st_ref, g_ref, b_ref), W_ref[...])
    x0_ref[...] = y
    _accum_stats(acc, y)

    @pl.when(pl.program_id(0) == NB - 1)
    def _():
        st1_ref[...] = _finalize_stats(acc[...], A_ref[...])


def _tc_stage1(x0_ref, st_ref, g_ref, b_ref, W_ref, A_ref,
               x1_ref, st2_ref, acc):
    y = _matmul_t(_gn_apply(x0_ref[...], st_ref, g_ref, b_ref), W_ref[...])
    x1_ref[...] = y
    _accum_stats(acc, y)

    @pl.when(pl.program_id(0) == NB - 1)
    def _():
        st2_ref[...] = _finalize_stats(acc[...], A_ref[...])


def _tc_stage2(x1_ref, lvcls_ref, st_ref, g_ref, b_ref, W_ref, tab_ref):
    big16 = _matmul_t(_gn_apply(x1_ref[...], st_ref, g_ref, b_ref), W_ref[...])
    tab_ref[:, 0:NC] = lvcls_ref[...]
    tab_ref[:, NC:TW] = big16


def _sc_body(tab_hbm, idx_hbm, baryT_hbm, scal_hbm, clsb_hbm,
             out_hbm,
             idx_v0, idx_v1, tab_v0, tab_v1, bary_v0, bary_v1,
             w_v, out_v0, out_v1, scal_v, clsb_v,
             sem_g0, sem_g1, sem_i0, sem_i1, sem_o0, sem_o1):
    cidx = lax.axis_index("c")
    sidx = lax.axis_index("s")
    nloc = jnp.where(cidx == 0, CH0, CHTOT - CH0)
    chunk0 = sidx * CHTOT + jnp.where(cidx == 0, 0, CH0)
    idx_v = [idx_v0, idx_v1]
    tab_v = [tab_v0, tab_v1]
    bary_v = [bary_v0, bary_v1]
    out_v = [out_v0, out_v1]
    sem_g = [sem_g0, sem_g1]
    sem_i = [sem_i0, sem_i1]
    sem_o = [sem_o0, sem_o1]

    pltpu.sync_copy(scal_hbm, scal_v)
    pltpu.sync_copy(clsb_hbm, clsb_v)
    sv = scal_v[...]
    lanes = lax.iota(jnp.int32, 16)

    def lane(k):
        return jnp.sum(jnp.where(lanes == k, sv, 0.0))

    u = [lane(j) for j in range(8)]
    u8 = lane(8)
    c0 = lane(9)
    one8 = lane(10)
    cb0 = clsb_v[pl.ds(0, 16)]
    cb1 = clsb_v[pl.ds(16, 16)]

    def idx_copy(g, b):
        return pltpu.make_async_copy(
            idx_hbm.at[pl.ds((chunk0 + g) * 4, 4)],
            idx_v[b], sem_i[b])

    def gather_copies(g, b):
        cps = [pltpu.make_async_copy(
            tab_hbm.at[idx_v[b].at[k]],
            tab_v[b].at[pl.ds(k * 128, 128)], sem_g[b]) for k in range(4)]
        cps.append(pltpu.make_async_copy(
            baryT_hbm.at[:, pl.ds((chunk0 + g) * CHUNK, CHUNK)],
            bary_v[b], sem_g[b]))
        return cps

    def out_copy(g, b):
        return pltpu.make_async_copy(
            out_v[b], out_hbm.at[pl.ds((chunk0 + g) * CHUNK, CHUNK)],
            sem_o[b])

    idx_copy(0, 0).start()
    idx_copy(0, 0).wait()
    for cp in gather_copies(0, 0):
        cp.start()
    idx_copy(1, 1).start()

    def compute(g, b):
        tabv = tab_v[b]
        barv = bary_v[b]
        outv = out_v[b]

        def groupA(gg, _):
            pos16 = gg * 16 + lanes
            rows = 4 * pos16
            bvec = [barv[i, pl.ds(gg * 16, 16)] for i in range(4)]
            mb = jnp.maximum(jnp.maximum(bvec[0], bvec[1]),
                             jnp.maximum(bvec[2], bvec[3]))
            tvec = []
            mj = [None] * 8
            for i in range(4):
                ri = rows + i
                tvec.append(plsc.load_gather(
                    tabv, [ri, jnp.full((16,), TCOL, jnp.int32)]))
                for j in range(8):
                    vij = plsc.load_gather(
                        tabv, [ri, jnp.full((16,), BN + j, jnp.int32)])
                    mj[j] = vij if i == 0 else jnp.maximum(mj[j], vij)
            s = u[0] * mj[0]
            for j in range(1, 8):
                s = s + u[j] * mj[j]
            Cp = c0 - s - u8 * mb
            for i in range(4):
                w_v[i, pl.ds(gg * 16, 16)] = one8 * bvec[i] + tvec[i] + Cp
            return 0

        lax.fori_loop(0, CHUNK // 16, groupA, 0, unroll=2)

        def posB(c, _):
            acc0 = cb0
            acc1 = cb1
            for i in range(4):
                fi = 4 * c + i
                wv = plsc.load_gather(
                    w_v, [jnp.full((16,), i, jnp.int32),
                          jnp.full((16,), c, jnp.int32)])
                acc0 = acc0 + wv * tabv[fi, pl.ds(0, 16)]
                acc1 = acc1 + wv * tabv[fi, pl.ds(16, 16)]
            outv[c, pl.ds(0, 16)] = acc0
            outv[c, pl.ds(16, 16)] = acc1
            return 0

        lax.fori_loop(0, CHUNK, posB, 0, unroll=4)

    def process(g, b):
        @pl.when(g + 1 < nloc)
        def _():
            idx_copy(g + 1, 1 - b).wait()
            for cp in gather_copies(g + 1, 1 - b):
                cp.start()

        @pl.when(g >= 2)
        def _():
            out_copy(g - 2, b).wait()

        for cp in gather_copies(g, b):
            cp.wait()
        compute(g, b)

        @pl.when(g + 2 < nloc)
        def _():
            idx_copy(g + 2, b).start()

        out_copy(g, b).start()

    def pair(gp, _):
        for par in range(2):
            g2 = 2 * gp + par

            @pl.when(g2 < nloc)
            def _():
                process(g2, par)
        return 0

    lax.fori_loop(0, (CHTOT - CH0 + 1) // 2, pair, 0)
    out_copy(nloc - 2, 0).wait()
    out_copy(nloc - 1, 1).wait()


def kernel(lv, splat_indices, barycentric, positions,
           gn0_gamma, gn0_beta, lin0_W,
           gn1_gamma, gn1_beta, lin1_W,
           gnb_gamma, gnb_beta, linb_W,
           gamma, beta, dW_W, dW_b, cls_W, cls_b):
    A0 = _group_avg_matrix(D, 32, V)
    A2 = _group_avg_matrix(D // 2, 32, V)
    Wbig = jnp.concatenate(
        [linb_W, (dW_W[0, :8] @ linb_W)[None, :],
         jnp.zeros((7, D // 2), jnp.float32)], axis=0)
    u = dW_W[0, :] * gamma
    c0 = dW_b[0] - jnp.dot(dW_W[0, :], beta)
    one8 = 1.0 + dW_W[0, 8]
    scal16 = jnp.concatenate(
        [u, c0[None], one8[None], jnp.zeros((5,), jnp.float32)])

    idx_pad = jnp.pad(splat_indices, ((0, PPAD - P), (0, 0)))
    idx2d = idx_pad.reshape(PPAD * NV // 128, 128)
    baryT = jnp.pad(barycentric, ((0, PPAD - P), (0, 0))).T

    def row(c):
        return pl.BlockSpec((VB, c), lambda i: (i, 0))

    def full(r, c):
        return pl.BlockSpec((r, c), lambda i: (0, 0))

    f32 = jnp.float32
    st0 = pl.pallas_call(
        _tc_stats0, grid=(NB,),
        in_specs=[row(D), full(D, D)],
        out_specs=full(2, D),
        out_shape=jax.ShapeDtypeStruct((2, D), f32),
        scratch_shapes=[pltpu.VMEM((2, D), f32)],
    )(lv, A0)

    x0, lvcls, st1 = pl.pallas_call(
        _tc_stage0, grid=(NB,),
        in_specs=[row(D), full(2, D), full(1, D), full(1, D),
                  full(D, D), full(NC, D), full(D, D)],
        out_specs=(row(D), row(NC), full(2, D)),
        out_shape=(jax.ShapeDtypeStruct((V, D), f32),
                   jax.ShapeDtypeStruct((V, NC), f32),
                   jax.ShapeDtypeStruct((2, D), f32)),
        scratch_shapes=[pltpu.VMEM((2, D), f32)],
    )(lv, st0, gn0_gamma.reshape(1, D), gn0_beta.reshape(1, D),
      lin0_W, cls_W, A0)

    x1, st2 = pl.pallas_call(
        _tc_stage1, grid=(NB,),
        in_specs=[row(D), full(2, D), full(1, D), full(1, D),
                  full(D // 2, D), full(D // 2, D // 2)],
        out_specs=(row(D // 2), full(2, D // 2)),
        out_shape=(jax.ShapeDtypeStruct((V, D // 2), f32),
                   jax.ShapeDtypeStruct((2, D // 2), f32)),
        scratch_shapes=[pltpu.VMEM((2, D // 2), f32)],
    )(x0, st1, gn1_gamma.reshape(1, D), gn1_beta.reshape(1, D),
      lin1_W, A2)

    tab128 = pl.pallas_call(
        _tc_stage2, grid=(NB,),
        in_specs=[row(D // 2), row(NC), full(2, D // 2), full(1, D // 2),
                  full(1, D // 2), full(16, D // 2)],
        out_specs=row(TW),
        out_shape=jax.ShapeDtypeStruct((V, TW), f32),
    )(x1, lvcls, st2, gnb_gamma.reshape(1, D // 2),
      gnb_beta.reshape(1, D // 2), Wbig)
    tab = tab128

    mesh = plsc.VectorSubcoreMesh(core_axis_name="c", subcore_axis_name="s")
    sc = pl.kernel(
        _sc_body, mesh=mesh,
        compiler_params=pltpu.CompilerParams(needs_layout_passes=False,
                                             use_tc_tiling_on_sc=False),
        out_type=jax.ShapeDtypeStruct((PPAD, NC), jnp.float32),
        scratch_types=[
            pltpu.VMEM((4, 128), jnp.int32),
            pltpu.VMEM((4, 128), jnp.int32),
            pltpu.VMEM((4 * CHUNK, TW), jnp.float32),
            pltpu.VMEM((4 * CHUNK, TW), jnp.float32),
            pltpu.VMEM((4, CHUNK), jnp.float32),
            pltpu.VMEM((4, CHUNK), jnp.float32),
            pltpu.VMEM((4, CHUNK), jnp.float32),
            pltpu.VMEM((CHUNK, NC), jnp.float32),
            pltpu.VMEM((CHUNK, NC), jnp.float32),
            pltpu.VMEM((16,), jnp.float32),
            pltpu.VMEM((NC,), jnp.float32),
            pltpu.SemaphoreType.DMA,
            pltpu.SemaphoreType.DMA,
            pltpu.SemaphoreType.DMA,
            pltpu.SemaphoreType.DMA,
            pltpu.SemaphoreType.DMA,
            pltpu.SemaphoreType.DMA,
        ])
    out_pad = sc(tab, idx2d, baryT, scal16, cls_b)
    return out_pad[:P]


# speedup vs baseline: 1.2866x; 1.2866x over previous
import jax
import jax.numpy as jnp
import numpy as np
from jax import lax
from jax.experimental import pallas as pl
from jax.experimental.pallas import tpu as pltpu
from jax.experimental.pallas import tpu_sc as plsc

V = 50000
P = 100000
D = 128
NV = 4
NC = 32
EPS = 1e-5

NW = 32
CHUNK = 128
TREAL = (P + CHUNK - 1) // CHUNK
TAIL = TREAL - 1
TAILN = P - TAIL * CHUNK
T0 = 32
T1 = 18
IDXROWS = TREAL * 4
TW = 48
BN = NC
TCOL = NC + 8

NB = 10
VB = V // NB


def _group_avg_matrix(C, G, Vn):
    gid = np.arange(C) // (C // G)
    A = (gid[:, None] == gid[None, :]).astype(np.float32) / (Vn * (C // G))
    return jnp.asarray(A)


def _matmul_t(x, W):
    return lax.dot_general(x, W, (((1,), (1,)), ((), ())),
                           preferred_element_type=jnp.float32)


def _finalize_stats(acc, A):
    meanb = jnp.dot(acc[0:1, :], A, preferred_element_type=jnp.float32)
    ex2b = jnp.dot(acc[1:2, :], A, preferred_element_type=jnp.float32)
    rstd = lax.rsqrt(ex2b - meanb * meanb + EPS)
    return jnp.concatenate([meanb, rstd], axis=0)


def _accum_stats(acc_ref, y):
    i = pl.program_id(0)

    @pl.when(i == 0)
    def _():
        acc_ref[...] = jnp.zeros_like(acc_ref)

    acc_ref[...] += jnp.concatenate(
        [jnp.sum(y, axis=0, keepdims=True),
         jnp.sum(y * y, axis=0, keepdims=True)], axis=0)


def _gn_apply(x, st_ref, g_ref, b_ref):
    xn = (x - st_ref[0:1, :]) * st_ref[1:2, :]
    return jnp.maximum(xn * g_ref[...] + b_ref[...], 0.0)


def _tc_stats0(lv_ref, A_ref, st_ref, acc):
    _accum_stats(acc, lv_ref[...])

    @pl.when(pl.program_id(0) == NB - 1)
    def _():
        st_ref[...] = _finalize_stats(acc[...], A_ref[...])


def _tc_stage0(lv_ref, st_ref, g_ref, b_ref, W_ref, cls_W_ref, A_ref,
               x0_ref, lvcls_ref, st1_ref, acc):
    lv = lv_ref[...]
    lvcls_ref[...] = _matmul_t(lv, cls_W_ref[...])
    y = _matmul_t(_gn_apply(lv, st_ref, g_ref, b_ref), W_ref[...])
    x0_ref[...] = y
    _accum_stats(acc, y)

    @pl.when(pl.program_id(0) == NB - 1)
    def _():
        st1_ref[...] = _finalize_stats(acc[...], A_ref[...])


def _tc_stage1(x0_ref, st_ref, g_ref, b_ref, W_ref, A_ref,
               x1_ref, st2_ref, acc):
    y = _matmul_t(_gn_apply(x0_ref[...], st_ref, g_ref, b_ref), W_ref[...])
    x1_ref[...] = y
    _accum_stats(acc, y)

    @pl.when(pl.program_id(0) == NB - 1)
    def _():
        st2_ref[...] = _finalize_stats(acc[...], A_ref[...])


def _tc_stage2(x1_ref, lvcls_ref, st_ref, g_ref, b_ref, W_ref, tab_ref):
    big16 = _matmul_t(_gn_apply(x1_ref[...], st_ref, g_ref, b_ref), W_ref[...])
    tab_ref[:, 0:NC] = lvcls_ref[...]
    tab_ref[:, NC:TW] = big16


def _sc_body(tab_hbm, idx_hbm, bary_hbm, scal_hbm, clsb_hbm,
             out_hbm,
             idx_v0, idx_v1, tab_v0, tab_v1, bary_v0, bary_v1,
             w_v, out_v0, out_v1, scal_v, clsb_v,
             sem_g0, sem_g1, sem_i0, sem_i1, sem_o0, sem_o1):
    cidx = lax.axis_index("c")
    sidx = lax.axis_index("s")
    chunk0 = jnp.where(cidx == 0, T0 * sidx, 16 * T0 + T1 * sidx)
    nloc = jnp.where(cidx == 0, T0,
                     jnp.maximum(0, jnp.minimum(T1, TREAL - chunk0)))
    idx_v = [idx_v0, idx_v1]
    tab_v = [tab_v0, tab_v1]
    bary_v = [bary_v0, bary_v1]
    out_v = [out_v0, out_v1]
    sem_g = [sem_g0, sem_g1]
    sem_i = [sem_i0, sem_i1]
    sem_o = [sem_o0, sem_o1]

    pltpu.sync_copy(scal_hbm, scal_v)
    pltpu.sync_copy(clsb_hbm, clsb_v)
    sv = scal_v[...]
    lanes = lax.iota(jnp.int32, 16)

    def lane(k):
        return jnp.sum(jnp.where(lanes == k, sv, 0.0))

    u = [lane(j) for j in range(8)]
    u8 = lane(8)
    c0 = lane(9)
    one8 = lane(10)
    cb0 = clsb_v[pl.ds(0, 16)]
    cb1 = clsb_v[pl.ds(16, 16)]

    def idx_copy(g, b):
        return pltpu.make_async_copy(
            idx_hbm.at[pl.ds((chunk0 + g) * 4, 4)],
            idx_v[b], sem_i[b])

    def gather_copies(g, b):
        cps = [pltpu.make_async_copy(
            tab_hbm.at[idx_v[b].at[k]],
            tab_v[b].at[pl.ds(k * 128, 128)], sem_g[b]) for k in range(4)]
        cps.append(pltpu.make_async_copy(
            bary_hbm.at[pl.ds((chunk0 + g) * 4, 4)],
            bary_v[b], sem_g[b]))
        return cps

    def out_full(g, b):
        return pltpu.make_async_copy(
            out_v[b], out_hbm.at[pl.ds((chunk0 + g) * CHUNK, CHUNK)],
            sem_o[b])

    def out_tail(b):
        return pltpu.make_async_copy(
            out_v[b].at[pl.ds(0, TAILN)],
            out_hbm.at[pl.ds(TAIL * CHUNK, TAILN)], sem_o[b])

    def out_start(g, b):
        @pl.when(chunk0 + g != TAIL)
        def _():
            out_full(g, b).start()

        @pl.when(chunk0 + g == TAIL)
        def _():
            out_tail(b).start()

    def out_wait(g, b):
        @pl.when(chunk0 + g != TAIL)
        def _():
            out_full(g, b).wait()

        @pl.when(chunk0 + g == TAIL)
        def _():
            out_tail(b).wait()

    @pl.when(nloc > 0)
    def _():
        idx_copy(0, 0).start()
        idx_copy(0, 0).wait()
        for cp in gather_copies(0, 0):
            cp.start()
        idx_copy(1, 1).start()

    def compute(g, b):
        tabv = tab_v[b]
        barv = bary_v[b]
        outv = out_v[b]

        def groupA(gg, _):
            pos16 = gg * 16 + lanes
            rows = 4 * pos16
            bvec = [plsc.load_gather(
                barv, [lax.shift_right_logical(rows + i, 7),
                       lax.bitwise_and(rows + i, 127)]) for i in range(4)]
            mb = jnp.maximum(jnp.maximum(bvec[0], bvec[1]),
                             jnp.maximum(bvec[2], bvec[3]))
            tvec = []
            mj = [None] * 8
            for i in range(4):
                ri = rows + i
                tvec.append(plsc.load_gather(
                    tabv, [ri, jnp.full((16,), TCOL, jnp.int32)]))
                for j in range(8):
                    vij = plsc.load_gather(
                        tabv, [ri, jnp.full((16,), BN + j, jnp.int32)])
                    mj[j] = vij if i == 0 else jnp.maximum(mj[j], vij)
            s = u[0] * mj[0]
            for j in range(1, 8):
                s = s + u[j] * mj[j]
            Cp = c0 - s - u8 * mb
            for i in range(4):
                w_v[i, pl.ds(gg * 16, 16)] = one8 * bvec[i] + tvec[i] + Cp
            return 0

        lax.fori_loop(0, CHUNK // 16, groupA, 0, unroll=2)

        def posB(c, _):
            acc0 = cb0
            acc1 = cb1
            for i in range(4):
                fi = 4 * c + i
                wv = plsc.load_gather(
                    w_v, [jnp.full((16,), i, jnp.int32),
                          jnp.full((16,), c, jnp.int32)])
                acc0 = acc0 + wv * tabv[fi, pl.ds(0, 16)]
                acc1 = acc1 + wv * tabv[fi, pl.ds(16, 16)]
            outv[c, pl.ds(0, 16)] = acc0
            outv[c, pl.ds(16, 16)] = acc1
            return 0

        lax.fori_loop(0, CHUNK, posB, 0, unroll=4)

    def process(g, b):
        @pl.when(g + 1 < nloc)
        def _():
            idx_copy(g + 1, 1 - b).wait()
            for cp in gather_copies(g + 1, 1 - b):
                cp.start()

        @pl.when(g >= 2)
        def _():
            out_wait(g - 2, b)

        for cp in gather_copies(g, b):
            cp.wait()
        compute(g, b)

        @pl.when(g + 2 < nloc)
        def _():
            idx_copy(g + 2, b).start()

        out_start(g, b)

    def pair(gp, _):
        for par in range(2):
            g2 = 2 * gp + par

            @pl.when(g2 < nloc)
            def _():
                process(g2, par)
        return 0

    lax.fori_loop(0, T0 // 2, pair, 0)

    @pl.when(nloc > 0)
    def _():
        out_wait(nloc - 2, 0)
        out_wait(nloc - 1, 1)


def kernel(lv, splat_indices, barycentric, positions,
           gn0_gamma, gn0_beta, lin0_W,
           gn1_gamma, gn1_beta, lin1_W,
           gnb_gamma, gnb_beta, linb_W,
           gamma, beta, dW_W, dW_b, cls_W, cls_b):
    A0 = _group_avg_matrix(D, 32, V)
    A2 = _group_avg_matrix(D // 2, 32, V)
    Wbig = jnp.concatenate(
        [linb_W, (dW_W[0, :8] @ linb_W)[None, :],
         jnp.zeros((7, D // 2), jnp.float32)], axis=0)
    u = dW_W[0, :] * gamma
    c0 = dW_b[0] - jnp.dot(dW_W[0, :], beta)
    one8 = 1.0 + dW_W[0, 8]
    scal16 = jnp.concatenate(
        [u, c0[None], one8[None], jnp.zeros((5,), jnp.float32)])

    idx2d = jnp.pad(splat_indices.reshape(P * NV // 128, 128), ((0, 3), (0, 0)))
    bary2d = jnp.pad(barycentric.reshape(P * NV // 128, 128), ((0, 3), (0, 0)))

    def row(c):
        return pl.BlockSpec((VB, c), lambda i: (i, 0))

    def full(r, c):
        return pl.BlockSpec((r, c), lambda i: (0, 0))

    f32 = jnp.float32
    st0 = pl.pallas_call(
        _tc_stats0, grid=(NB,),
        in_specs=[row(D), full(D, D)],
        out_specs=full(2, D),
        out_shape=jax.ShapeDtypeStruct((2, D), f32),
        scratch_shapes=[pltpu.VMEM((2, D), f32)],
    )(lv, A0)

    x0, lvcls, st1 = pl.pallas_call(
        _tc_stage0, grid=(NB,),
        in_specs=[row(D), full(2, D), full(1, D), full(1, D),
                  full(D, D), full(NC, D), full(D, D)],
        out_specs=(row(D), row(NC), full(2, D)),
        out_shape=(jax.ShapeDtypeStruct((V, D), f32),
                   jax.ShapeDtypeStruct((V, NC), f32),
                   jax.ShapeDtypeStruct((2, D), f32)),
        scratch_shapes=[pltpu.VMEM((2, D), f32)],
    )(lv, st0, gn0_gamma.reshape(1, D), gn0_beta.reshape(1, D),
      lin0_W, cls_W, A0)

    x1, st2 = pl.pallas_call(
        _tc_stage1, grid=(NB,),
        in_specs=[row(D), full(2, D), full(1, D), full(1, D),
                  full(D // 2, D), full(D // 2, D // 2)],
        out_specs=(row(D // 2), full(2, D // 2)),
        out_shape=(jax.ShapeDtypeStruct((V, D // 2), f32),
                   jax.ShapeDtypeStruct((2, D // 2), f32)),
        scratch_shapes=[pltpu.VMEM((2, D // 2), f32)],
    )(x0, st1, gn1_gamma.reshape(1, D), gn1_beta.reshape(1, D),
      lin1_W, A2)

    tab128 = pl.pallas_call(
        _tc_stage2, grid=(NB,),
        in_specs=[row(D // 2), row(NC), full(2, D // 2), full(1, D // 2),
                  full(1, D // 2), full(16, D // 2)],
        out_specs=row(TW),
        out_shape=jax.ShapeDtypeStruct((V, TW), f32),
    )(x1, lvcls, st2, gnb_gamma.reshape(1, D // 2),
      gnb_beta.reshape(1, D // 2), Wbig)
    tab = tab128

    mesh = plsc.VectorSubcoreMesh(core_axis_name="c", subcore_axis_name="s")
    sc = pl.kernel(
        _sc_body, mesh=mesh,
        compiler_params=pltpu.CompilerParams(needs_layout_passes=False,
                                             use_tc_tiling_on_sc=False),
        out_type=jax.ShapeDtypeStruct((P, NC), jnp.float32),
        scratch_types=[
            pltpu.VMEM((4, 128), jnp.int32),
            pltpu.VMEM((4, 128), jnp.int32),
            pltpu.VMEM((4 * CHUNK, TW), jnp.float32),
            pltpu.VMEM((4 * CHUNK, TW), jnp.float32),
            pltpu.VMEM((4, CHUNK), jnp.float32),
            pltpu.VMEM((4, CHUNK), jnp.float32),
            pltpu.VMEM((4, CHUNK), jnp.float32),
            pltpu.VMEM((CHUNK, NC), jnp.float32),
            pltpu.VMEM((CHUNK, NC), jnp.float32),
            pltpu.VMEM((16,), jnp.float32),
            pltpu.VMEM((NC,), jnp.float32),
            pltpu.SemaphoreType.DMA,
            pltpu.SemaphoreType.DMA,
            pltpu.SemaphoreType.DMA,
            pltpu.SemaphoreType.DMA,
            pltpu.SemaphoreType.DMA,
            pltpu.SemaphoreType.DMA,
        ])
    return sc(tab, idx2d, bary2d, scal16, cls_b)


# speedup vs baseline: 1.4033x; 1.0907x over previous
import jax
import jax.numpy as jnp
import numpy as np
from jax import lax
from jax.experimental import pallas as pl
from jax.experimental.pallas import tpu as pltpu
from jax.experimental.pallas import tpu_sc as plsc

V = 50000
P = 100000
D = 128
NV = 4
NC = 32
EPS = 1e-5

NW = 32
CHUNK = 128
TREAL = (P + CHUNK - 1) // CHUNK
TAIL = TREAL - 1
TAILN = P - TAIL * CHUNK
T0 = 32
T1 = 18
IDXROWS = TREAL * 4
TW = 48
BN = NC
TCOL = NC + 8

NB = 10
VB = V // NB


def _group_avg_matrix(C, G, Vn):
    gid = np.arange(C) // (C // G)
    A = (gid[:, None] == gid[None, :]).astype(np.float32) / (Vn * (C // G))
    return jnp.asarray(A)


def _matmul_t(x, W):
    return lax.dot_general(x, W, (((1,), (1,)), ((), ())),
                           preferred_element_type=jnp.float32)


def _finalize_stats(acc, A):
    meanb = jnp.dot(acc[0:1, :], A, preferred_element_type=jnp.float32)
    ex2b = jnp.dot(acc[1:2, :], A, preferred_element_type=jnp.float32)
    rstd = lax.rsqrt(ex2b - meanb * meanb + EPS)
    return jnp.concatenate([meanb, rstd], axis=0)


def _accum_stats(acc_ref, y):
    i = pl.program_id(0)

    @pl.when(i == 0)
    def _():
        acc_ref[...] = jnp.zeros_like(acc_ref)

    acc_ref[...] += jnp.concatenate(
        [jnp.sum(y, axis=0, keepdims=True),
         jnp.sum(y * y, axis=0, keepdims=True)], axis=0)


def _gn_apply(x, st_ref, g_ref, b_ref):
    xn = (x - st_ref[0:1, :]) * st_ref[1:2, :]
    return jnp.maximum(xn * g_ref[...] + b_ref[...], 0.0)


def _tc_stats0(lv_ref, A_ref, st_ref, acc):
    _accum_stats(acc, lv_ref[...])

    @pl.when(pl.program_id(0) == NB - 1)
    def _():
        st_ref[...] = _finalize_stats(acc[...], A_ref[...])


def _tc_stage0(lv_ref, st_ref, g_ref, b_ref, W_ref, cls_W_ref, A_ref,
               x0_ref, lvcls_ref, st1_ref, acc):
    lv = lv_ref[...]
    lvcls_ref[...] = _matmul_t(lv, cls_W_ref[...])
    y = _matmul_t(_gn_apply(lv, st_ref, g_ref, b_ref), W_ref[...])
    x0_ref[...] = y
    _accum_stats(acc, y)

    @pl.when(pl.program_id(0) == NB - 1)
    def _():
        st1_ref[...] = _finalize_stats(acc[...], A_ref[...])


def _tc_stage1(x0_ref, st_ref, g_ref, b_ref, W_ref, A_ref,
               x1_ref, st2_ref, acc):
    y = _matmul_t(_gn_apply(x0_ref[...], st_ref, g_ref, b_ref), W_ref[...])
    x1_ref[...] = y
    _accum_stats(acc, y)

    @pl.when(pl.program_id(0) == NB - 1)
    def _():
        st2_ref[...] = _finalize_stats(acc[...], A_ref[...])


PB = 9984
NPB = 11
IBW = NPB * PB


def _tc_transpose(idx_ref, bary_ref, ib_ref):
    idxT = jnp.transpose(idx_ref[...], (1, 0))
    idxT = jnp.clip(idxT, 0, V - 1)
    baryT = jnp.transpose(bary_ref[...], (1, 0))
    ib_ref[...] = jnp.concatenate(
        [idxT, lax.bitcast_convert_type(baryT, jnp.int32)], axis=0)


def _tc_stage2(x1_ref, lvcls_ref, st_ref, g_ref, b_ref, W_ref, tab_ref):
    big16 = _matmul_t(_gn_apply(x1_ref[...], st_ref, g_ref, b_ref), W_ref[...])
    tab_ref[:, 0:NC] = lvcls_ref[...]
    tab_ref[:, NC:TW] = big16


def _sc_body(tab_hbm, ib_hbm, scal_hbm, clsb_hbm,
             out_hbm,
             ib_v0, ib_v1, tab_v0, tab_v1,
             w_v, out_v0, out_v1, scal_v, clsb_v,
             sem_g0, sem_g1, sem_i0, sem_i1, sem_o0, sem_o1):
    cidx = lax.axis_index("c")
    sidx = lax.axis_index("s")
    chunk0 = jnp.where(cidx == 0, T0 * sidx, 16 * T0 + T1 * sidx)
    nloc = jnp.where(cidx == 0, T0,
                     jnp.maximum(0, jnp.minimum(T1, TREAL - chunk0)))
    ib_v = [ib_v0, ib_v1]
    tab_v = [tab_v0, tab_v1]
    out_v = [out_v0, out_v1]
    sem_g = [sem_g0, sem_g1]
    sem_i = [sem_i0, sem_i1]
    sem_o = [sem_o0, sem_o1]

    pltpu.sync_copy(scal_hbm, scal_v)
    pltpu.sync_copy(clsb_hbm, clsb_v)
    sv = scal_v[...]
    lanes = lax.iota(jnp.int32, 16)

    def lane(k):
        return jnp.sum(jnp.where(lanes == k, sv, 0.0))

    u = [lane(j) for j in range(8)]
    u8 = lane(8)
    c0 = lane(9)
    one8 = lane(10)
    cb0 = clsb_v[pl.ds(0, 16)]
    cb1 = clsb_v[pl.ds(16, 16)]

    def idx_copy(g, b):
        return pltpu.make_async_copy(ib_hbm.at[chunk0 + g], ib_v[b], sem_i[b])

    def gather_copies(g, b):
        return [pltpu.make_async_copy(
            tab_hbm.at[ib_v[b].at[k]],
            tab_v[b].at[pl.ds(k * 128, 128)], sem_g[b]) for k in range(4)]

    def out_full(g, b):
        return pltpu.make_async_copy(
            out_v[b], out_hbm.at[pl.ds((chunk0 + g) * CHUNK, CHUNK)],
            sem_o[b])

    def out_tail(b):
        return pltpu.make_async_copy(
            out_v[b].at[pl.ds(0, TAILN)],
            out_hbm.at[pl.ds(TAIL * CHUNK, TAILN)], sem_o[b])

    def out_start(g, b):
        @pl.when(chunk0 + g != TAIL)
        def _():
            out_full(g, b).start()

        @pl.when(chunk0 + g == TAIL)
        def _():
            out_tail(b).start()

    def out_wait(g, b):
        @pl.when(chunk0 + g != TAIL)
        def _():
            out_full(g, b).wait()

        @pl.when(chunk0 + g == TAIL)
        def _():
            out_tail(b).wait()

    @pl.when(nloc > 0)
    def _():
        idx_copy(0, 0).start()
        idx_copy(0, 0).wait()
        for cp in gather_copies(0, 0):
            cp.start()
        idx_copy(1, 1).start()

    def compute(g, b):
        tabv = tab_v[b]
        ibv = ib_v[b]
        outv = out_v[b]

        def groupA(gg, _):
            pos16 = gg * 16 + lanes
            rows = 4 * pos16
            bvec = [plsc.bitcast(ibv[4 + i, pl.ds(gg * 16, 16)], jnp.float32)
                    for i in range(4)]
            mb = jnp.maximum(jnp.maximum(bvec[0], bvec[1]),
                             jnp.maximum(bvec[2], bvec[3]))
            tvec = []
            mj = [None] * 8
            for i in range(4):
                ri = rows + i
                tvec.append(plsc.load_gather(
                    tabv, [ri, jnp.full((16,), TCOL, jnp.int32)]))
                for j in range(8):
                    vij = plsc.load_gather(
                        tabv, [ri, jnp.full((16,), BN + j, jnp.int32)])
                    mj[j] = vij if i == 0 else jnp.maximum(mj[j], vij)
            s = u[0] * mj[0]
            for j in range(1, 8):
                s = s + u[j] * mj[j]
            Cp = c0 - s - u8 * mb
            for i in range(4):
                w_v[i, pl.ds(gg * 16, 16)] = one8 * bvec[i] + tvec[i] + Cp
            return 0

        lax.fori_loop(0, CHUNK // 16, groupA, 0, unroll=2)

        def posB(c, _):
            acc0 = cb0
            acc1 = cb1
            for i in range(4):
                fi = 4 * c + i
                wv = plsc.load_gather(
                    w_v, [jnp.full((16,), i, jnp.int32),
                          jnp.full((16,), c, jnp.int32)])
                acc0 = acc0 + wv * tabv[fi, pl.ds(0, 16)]
                acc1 = acc1 + wv * tabv[fi, pl.ds(16, 16)]
            outv[c, pl.ds(0, 16)] = acc0
            outv[c, pl.ds(16, 16)] = acc1
            return 0

        lax.fori_loop(0, CHUNK, posB, 0, unroll=4)

    def process(g, b):
        @pl.when(g + 1 < nloc)
        def _():
            idx_copy(g + 1, 1 - b).wait()
            for cp in gather_copies(g + 1, 1 - b):
                cp.start()

        @pl.when(g >= 2)
        def _():
            out_wait(g - 2, b)

        for cp in gather_copies(g, b):
            cp.wait()
        compute(g, b)

        @pl.when(g + 2 < nloc)
        def _():
            idx_copy(g + 2, b).start()

        out_start(g, b)

    def pair(gp, _):
        for par in range(2):
            g2 = 2 * gp + par

            @pl.when(g2 < nloc)
            def _():
                process(g2, par)
        return 0

    lax.fori_loop(0, T0 // 2, pair, 0)

    @pl.when(nloc > 0)
    def _():
        out_wait(nloc - 2, 0)
        out_wait(nloc - 1, 1)


def kernel(lv, splat_indices, barycentric, positions,
           gn0_gamma, gn0_beta, lin0_W,
           gn1_gamma, gn1_beta, lin1_W,
           gnb_gamma, gnb_beta, linb_W,
           gamma, beta, dW_W, dW_b, cls_W, cls_b):
    A0 = _group_avg_matrix(D, 32, V)
    A2 = _group_avg_matrix(D // 2, 32, V)
    Wbig = jnp.concatenate(
        [linb_W, (dW_W[0, :8] @ linb_W)[None, :],
         jnp.zeros((7, D // 2), jnp.float32)], axis=0)
    u = dW_W[0, :] * gamma
    c0 = dW_b[0] - jnp.dot(dW_W[0, :], beta)
    one8 = 1.0 + dW_W[0, 8]
    scal16 = jnp.concatenate(
        [u, c0[None], one8[None], jnp.zeros((5,), jnp.float32)])


    def row(c):
        return pl.BlockSpec((VB, c), lambda i: (i, 0))

    def full(r, c):
        return pl.BlockSpec((r, c), lambda i: (0, 0))

    f32 = jnp.float32
    ib = pl.pallas_call(
        _tc_transpose, grid=(NPB,),
        in_specs=[pl.BlockSpec((PB, NV), lambda i: (i, 0)),
                  pl.BlockSpec((PB, NV), lambda i: (i, 0))],
        out_specs=pl.BlockSpec((8, PB), lambda i: (0, i)),
        out_shape=jax.ShapeDtypeStruct((8, IBW), jnp.int32),
    )(splat_indices, barycentric)
    ib3d = jnp.transpose(ib.reshape(8, IBW // 128, 128), (1, 0, 2))

    st0 = pl.pallas_call(
        _tc_stats0, grid=(NB,),
        in_specs=[row(D), full(D, D)],
        out_specs=full(2, D),
        out_shape=jax.ShapeDtypeStruct((2, D), f32),
        scratch_shapes=[pltpu.VMEM((2, D), f32)],
    )(lv, A0)

    x0, lvcls, st1 = pl.pallas_call(
        _tc_stage0, grid=(NB,),
        in_specs=[row(D), full(2, D), full(1, D), full(1, D),
                  full(D, D), full(NC, D), full(D, D)],
        out_specs=(row(D), row(NC), full(2, D)),
        out_shape=(jax.ShapeDtypeStruct((V, D), f32),
                   jax.ShapeDtypeStruct((V, NC), f32),
                   jax.ShapeDtypeStruct((2, D), f32)),
        scratch_shapes=[pltpu.VMEM((2, D), f32)],
    )(lv, st0, gn0_gamma.reshape(1, D), gn0_beta.reshape(1, D),
      lin0_W, cls_W, A0)

    x1, st2 = pl.pallas_call(
        _tc_stage1, grid=(NB,),
        in_specs=[row(D), full(2, D), full(1, D), full(1, D),
                  full(D // 2, D), full(D // 2, D // 2)],
        out_specs=(row(D // 2), full(2, D // 2)),
        out_shape=(jax.ShapeDtypeStruct((V, D // 2), f32),
                   jax.ShapeDtypeStruct((2, D // 2), f32)),
        scratch_shapes=[pltpu.VMEM((2, D // 2), f32)],
    )(x0, st1, gn1_gamma.reshape(1, D), gn1_beta.reshape(1, D),
      lin1_W, A2)

    tab128 = pl.pallas_call(
        _tc_stage2, grid=(NB,),
        in_specs=[row(D // 2), row(NC), full(2, D // 2), full(1, D // 2),
                  full(1, D // 2), full(16, D // 2)],
        out_specs=row(TW),
        out_shape=jax.ShapeDtypeStruct((V, TW), f32),
    )(x1, lvcls, st2, gnb_gamma.reshape(1, D // 2),
      gnb_beta.reshape(1, D // 2), Wbig)
    tab = tab128

    mesh = plsc.VectorSubcoreMesh(core_axis_name="c", subcore_axis_name="s")
    sc = pl.kernel(
        _sc_body, mesh=mesh,
        compiler_params=pltpu.CompilerParams(needs_layout_passes=False,
                                             use_tc_tiling_on_sc=False),
        out_type=jax.ShapeDtypeStruct((P, NC), jnp.float32),
        scratch_types=[
            pltpu.VMEM((8, 128), jnp.int32),
            pltpu.VMEM((8, 128), jnp.int32),
            pltpu.VMEM((4 * CHUNK, TW), jnp.float32),
            pltpu.VMEM((4 * CHUNK, TW), jnp.float32),
            pltpu.VMEM((4, CHUNK), jnp.float32),
            pltpu.VMEM((CHUNK, NC), jnp.float32),
            pltpu.VMEM((CHUNK, NC), jnp.float32),
            pltpu.VMEM((16,), jnp.float32),
            pltpu.VMEM((NC,), jnp.float32),
            pltpu.SemaphoreType.DMA,
            pltpu.SemaphoreType.DMA,
            pltpu.SemaphoreType.DMA,
            pltpu.SemaphoreType.DMA,
            pltpu.SemaphoreType.DMA,
            pltpu.SemaphoreType.DMA,
        ])
    return sc(tab, ib3d, scal16, cls_b)
